# initial kernel scaffold (unmeasured)
import functools

import jax
import jax.numpy as jnp
from jax import lax
from jax.experimental import pallas as pl
from jax.experimental.pallas import tpu as pltpu

B, S, H, Dh, Dr = 1, 1024, 16, 128, 32
D = 2048
DC = 256
DC_SH = 128
SCALE = (Dh + Dr) ** -0.5
F32 = jnp.float32


def _comm_body(x_ref, wdkv_ref, wuk_ref, wuv_ref,
               c_ref, wukf_ref, wuvf_ref,
               send_sems, recv_sems):
    my_x = lax.axis_index("x")
    my_y = lax.axis_index("y")
    peer = (my_x, 1 - my_y)
    off = my_y * DC_SH

    cpart = jnp.dot(x_ref[...], wdkv_ref[...], preferred_element_type=F32)
    c_ref[:, pl.ds(off, DC_SH)] = cpart
    wukf_ref[pl.ds(off, DC_SH), :] = wuk_ref[...]
    wuvf_ref[pl.ds(off, DC_SH), :] = wuv_ref[...]

    barrier = pltpu.get_barrier_semaphore()
    pl.semaphore_signal(barrier, inc=1, device_id=peer,
                        device_id_type=pl.DeviceIdType.MESH)
    pl.semaphore_wait(barrier, 1)

    srcs = [
        c_ref.at[:, pl.ds(off, DC_SH)],
        wukf_ref.at[pl.ds(off, DC_SH), :],
        wuvf_ref.at[pl.ds(off, DC_SH), :],
    ]
    rdmas = []
    for i, src in enumerate(srcs):
        rdma = pltpu.make_async_remote_copy(
            src_ref=src, dst_ref=src,
            send_sem=send_sems.at[i], recv_sem=recv_sems.at[i],
            device_id=peer, device_id_type=pl.DeviceIdType.MESH,
        )
        rdma.start()
        rdmas.append(rdma)
    for rdma in rdmas:
        rdma.wait()


def _comm(x2d, wdkv_sh, wuk_sh, wuv_sh):
    return pl.pallas_call(
        _comm_body,
        out_shape=(
            jax.ShapeDtypeStruct((S, DC), F32),
            jax.ShapeDtypeStruct((DC, D), F32),
            jax.ShapeDtypeStruct((DC, D), F32),
        ),
        in_specs=[pl.BlockSpec(memory_space=pltpu.VMEM)] * 4,
        out_specs=(pl.BlockSpec(memory_space=pltpu.VMEM),) * 3,
        scratch_shapes=[
            pltpu.SemaphoreType.DMA((3,)),
            pltpu.SemaphoreType.DMA((3,)),
        ],
        compiler_params=pltpu.CompilerParams(collective_id=0),
    )(x2d, wdkv_sh, wuk_sh, wuv_sh)


def _qkv_body(x_ref, wq_ref, wqr_ref, wkr_ref, q_ref, qr_ref, kr_ref):
    x = x_ref[...]
    q_ref[...] = jnp.dot(x, wq_ref[...], preferred_element_type=F32)
    qr_ref[...] = jnp.dot(x, wqr_ref[...], preferred_element_type=F32)
    kr_ref[...] = jnp.dot(x, wkr_ref[...], preferred_element_type=F32)


def _qkv(x2d, wq, wqr, wkr):
    return pl.pallas_call(
        _qkv_body,
        out_shape=(
            jax.ShapeDtypeStruct((S, D), F32),
            jax.ShapeDtypeStruct((S, H * Dr), F32),
            jax.ShapeDtypeStruct((S, Dr), F32),
        ),
        in_specs=[pl.BlockSpec(memory_space=pltpu.VMEM)] * 4,
        out_specs=(pl.BlockSpec(memory_space=pltpu.VMEM),) * 3,
    )(x2d, wq, wqr, wkr)


def _kv_body(c_ref, wuk_ref, wuv_ref, k_ref, v_ref):
    c = c_ref[...]
    k_ref[...] = jnp.dot(c, wuk_ref[...], preferred_element_type=F32)
    v_ref[...] = jnp.dot(c, wuv_ref[...], preferred_element_type=F32)


def _kv(c, wukf, wuvf):
    return pl.pallas_call(
        _kv_body,
        out_shape=(
            jax.ShapeDtypeStruct((S, D), F32),
            jax.ShapeDtypeStruct((S, D), F32),
        ),
        in_specs=[pl.BlockSpec(memory_space=pltpu.VMEM)] * 3,
        out_specs=(pl.BlockSpec(memory_space=pltpu.VMEM),) * 2,
    )(c, wukf, wuvf)


def _attn_body(q_ref, qr_ref, kr_ref, k_ref, v_ref, o_ref):
    s = lax.dot_general(q_ref[...], k_ref[...],
                        (((1,), (1,)), ((), ())),
                        preferred_element_type=F32)
    s = s + lax.dot_general(qr_ref[...], kr_ref[...],
                            (((1,), (1,)), ((), ())),
                            preferred_element_type=F32)
    s = s * SCALE
    m = jnp.max(s, axis=1, keepdims=True)
    p = jnp.exp(s - m)
    p = p / jnp.sum(p, axis=1, keepdims=True)
    o_ref[...] = lax.dot_general(p, v_ref[...],
                                 (((1,), (0,)), ((), ())),
                                 preferred_element_type=F32)


def _attn(q, qr, kr, k, v):
    return pl.pallas_call(
        _attn_body,
        grid=(H,),
        in_specs=[
            pl.BlockSpec((S, Dh), lambda h: (0, h)),
            pl.BlockSpec((S, Dr), lambda h: (0, h)),
            pl.BlockSpec((S, Dr), lambda h: (0, 0)),
            pl.BlockSpec((S, Dh), lambda h: (0, h)),
            pl.BlockSpec((S, Dh), lambda h: (0, h)),
        ],
        out_specs=pl.BlockSpec((S, Dh), lambda h: (0, h)),
        out_shape=jax.ShapeDtypeStruct((S, D), F32),
    )(q, qr, kr, k, v)


def _proj_body(o_ref, wo_ref, out_ref):
    out_ref[...] = jnp.dot(o_ref[...], wo_ref[...],
                           preferred_element_type=F32)


def _proj(o, wo):
    return pl.pallas_call(
        _proj_body,
        out_shape=jax.ShapeDtypeStruct((S, D), F32),
        in_specs=[pl.BlockSpec(memory_space=pltpu.VMEM)] * 2,
        out_specs=pl.BlockSpec(memory_space=pltpu.VMEM),
    )(o, wo)


def kernel(x, Wdkv, Wuk, Wuv, Wq, Wqr, Wkr, Wo):
    x2d = x.reshape(S, D)
    c, wukf, wuvf = _comm(x2d, Wdkv, Wuk, Wuv)
    q, qr, kr = _qkv(x2d, Wq, Wqr, Wkr)
    k, v = _kv(c, wukf, wuvf)
    o = _attn(q, qr, kr, k, v)
    out = _proj(o, Wo)
    return out.reshape(B, S, D)


# baseline (device time: 158328 ns/iter reference)
import functools

import jax
import jax.numpy as jnp
from jax import lax
from jax.experimental import pallas as pl
from jax.experimental.pallas import tpu as pltpu

B, S, H, Dh, Dr = 1, 1024, 16, 128, 32
D = 2048
DC = 256
DC_SH = 128
SCALE = (Dh + Dr) ** -0.5
F32 = jnp.float32


def _comm_body(x_ref, wdkv_ref, wuk_ref, wuv_ref,
               c_ref, wukf_ref, wuvf_ref,
               send_sems, recv_sems):
    my_x = lax.axis_index("x")
    my_y = lax.axis_index("y")
    peer = (my_x, 1 - my_y)
    off = my_y * DC_SH

    cpart = jnp.dot(x_ref[...], wdkv_ref[...], preferred_element_type=F32)
    c_ref[:, pl.ds(off, DC_SH)] = cpart
    wukf_ref[pl.ds(off, DC_SH), :] = wuk_ref[...]
    wuvf_ref[pl.ds(off, DC_SH), :] = wuv_ref[...]

    barrier = pltpu.get_barrier_semaphore()
    pl.semaphore_signal(barrier, inc=1, device_id=peer,
                        device_id_type=pl.DeviceIdType.MESH)
    pl.semaphore_wait(barrier, 1)

    srcs = [
        c_ref.at[:, pl.ds(off, DC_SH)],
        wukf_ref.at[pl.ds(off, DC_SH), :],
        wuvf_ref.at[pl.ds(off, DC_SH), :],
    ]
    rdmas = []
    for i, src in enumerate(srcs):
        rdma = pltpu.make_async_remote_copy(
            src_ref=src, dst_ref=src,
            send_sem=send_sems.at[i], recv_sem=recv_sems.at[i],
            device_id=peer, device_id_type=pl.DeviceIdType.MESH,
        )
        rdma.start()
        rdmas.append(rdma)
    for rdma in rdmas:
        rdma.wait()


def _comm(x2d, wdkv_sh, wuk_sh, wuv_sh):
    return pl.pallas_call(
        _comm_body,
        out_shape=(
            jax.ShapeDtypeStruct((S, DC), F32),
            jax.ShapeDtypeStruct((DC, D), F32),
            jax.ShapeDtypeStruct((DC, D), F32),
        ),
        in_specs=[pl.BlockSpec(memory_space=pltpu.VMEM)] * 4,
        out_specs=(pl.BlockSpec(memory_space=pltpu.VMEM),) * 3,
        scratch_shapes=[
            pltpu.SemaphoreType.DMA((3,)),
            pltpu.SemaphoreType.DMA((3,)),
        ],
        compiler_params=pltpu.CompilerParams(collective_id=0),
    )(x2d, wdkv_sh, wuk_sh, wuv_sh)


def _qkv_body(x_ref, wq_ref, wqr_ref, wkr_ref, q_ref, qr_ref, kr_ref):
    x = x_ref[...]
    q_ref[...] = jnp.dot(x, wq_ref[...], preferred_element_type=F32)
    qr_ref[...] = jnp.dot(x, wqr_ref[...], preferred_element_type=F32)
    kr_ref[...] = jnp.dot(x, wkr_ref[...], preferred_element_type=F32)


def _qkv(x2d, wq, wqr, wkr):
    return pl.pallas_call(
        _qkv_body,
        out_shape=(
            jax.ShapeDtypeStruct((S, D), F32),
            jax.ShapeDtypeStruct((S, H * Dr), F32),
            jax.ShapeDtypeStruct((S, Dr), F32),
        ),
        in_specs=[pl.BlockSpec(memory_space=pltpu.VMEM)] * 4,
        out_specs=(pl.BlockSpec(memory_space=pltpu.VMEM),) * 3,
    )(x2d, wq, wqr, wkr)


def _kv_body(c_ref, wuk_ref, wuv_ref, k_ref, v_ref):
    c = c_ref[...]
    k_ref[...] = jnp.dot(c, wuk_ref[...], preferred_element_type=F32)
    v_ref[...] = jnp.dot(c, wuv_ref[...], preferred_element_type=F32)


def _kv(c, wukf, wuvf):
    return pl.pallas_call(
        _kv_body,
        out_shape=(
            jax.ShapeDtypeStruct((S, D), F32),
            jax.ShapeDtypeStruct((S, D), F32),
        ),
        in_specs=[pl.BlockSpec(memory_space=pltpu.VMEM)] * 3,
        out_specs=(pl.BlockSpec(memory_space=pltpu.VMEM),) * 2,
    )(c, wukf, wuvf)


def _attn_body(q_ref, qr_ref, kr_ref, k_ref, v_ref, o_ref):
    s = lax.dot_general(q_ref[...], k_ref[...],
                        (((1,), (1,)), ((), ())),
                        preferred_element_type=F32)
    s = s + lax.dot_general(qr_ref[0], kr_ref[...],
                            (((1,), (1,)), ((), ())),
                            preferred_element_type=F32)
    s = s * SCALE
    m = jnp.max(s, axis=1, keepdims=True)
    p = jnp.exp(s - m)
    p = p / jnp.sum(p, axis=1, keepdims=True)
    o_ref[...] = lax.dot_general(p, v_ref[...],
                                 (((1,), (0,)), ((), ())),
                                 preferred_element_type=F32)


def _attn(q, qr3, kr, k, v):
    return pl.pallas_call(
        _attn_body,
        grid=(H,),
        in_specs=[
            pl.BlockSpec((S, Dh), lambda h: (0, h)),
            pl.BlockSpec((1, S, Dr), lambda h: (h, 0, 0)),
            pl.BlockSpec((S, Dr), lambda h: (0, 0)),
            pl.BlockSpec((S, Dh), lambda h: (0, h)),
            pl.BlockSpec((S, Dh), lambda h: (0, h)),
        ],
        out_specs=pl.BlockSpec((S, Dh), lambda h: (0, h)),
        out_shape=jax.ShapeDtypeStruct((S, D), F32),
    )(q, qr3, kr, k, v)


def _proj_body(o_ref, wo_ref, out_ref):
    out_ref[...] = jnp.dot(o_ref[...], wo_ref[...],
                           preferred_element_type=F32)


def _proj(o, wo):
    return pl.pallas_call(
        _proj_body,
        out_shape=jax.ShapeDtypeStruct((S, D), F32),
        in_specs=[pl.BlockSpec(memory_space=pltpu.VMEM)] * 2,
        out_specs=pl.BlockSpec(memory_space=pltpu.VMEM),
    )(o, wo)


def kernel(x, Wdkv, Wuk, Wuv, Wq, Wqr, Wkr, Wo):
    x2d = x.reshape(S, D)
    c, wukf, wuvf = _comm(x2d, Wdkv, Wuk, Wuv)
    q, qr, kr = _qkv(x2d, Wq, Wqr, Wkr)
    k, v = _kv(c, wukf, wuvf)
    qr3 = qr.reshape(S, H, Dr).transpose(1, 0, 2)
    o = _attn(q, qr3, kr, k, v)
    out = _proj(o, Wo)
    return out.reshape(B, S, D)


# device time: 149004 ns/iter; 1.0626x vs baseline; 1.0626x over previous
import jax
import jax.numpy as jnp
from jax import lax
from jax.experimental import pallas as pl
from jax.experimental.pallas import tpu as pltpu

B, S, H, Dh, Dr = 1, 1024, 16, 128, 32
D = 2048
DC = 256
DC_SH = 128
SCALE = (Dh + Dr) ** -0.5
F32 = jnp.float32


def _proj_comm_body(x_ref, wdkv_ref, wuk_ref, wuv_ref, wq_ref, wqr_ref,
                    wkr_ref,
                    q_ref, qr_ref, kr_ref, c_ref, wukf_ref, wuvf_ref,
                    send_sems, recv_sems):
    my_x = lax.axis_index("x")
    my_y = lax.axis_index("y")
    peer = (my_x, 1 - my_y)
    off = my_y * DC_SH

    x = x_ref[...]
    cpart = jnp.dot(x, wdkv_ref[...], preferred_element_type=F32)
    c_ref[:, pl.ds(off, DC_SH)] = cpart
    wukf_ref[pl.ds(off, DC_SH), :] = wuk_ref[...]
    wuvf_ref[pl.ds(off, DC_SH), :] = wuv_ref[...]

    barrier = pltpu.get_barrier_semaphore()
    pl.semaphore_signal(barrier, inc=1, device_id=peer,
                        device_id_type=pl.DeviceIdType.MESH)
    pl.semaphore_wait(barrier, 1)

    srcs = [
        c_ref.at[:, pl.ds(off, DC_SH)],
        wukf_ref.at[pl.ds(off, DC_SH), :],
        wuvf_ref.at[pl.ds(off, DC_SH), :],
    ]
    rdmas = []
    for i, src in enumerate(srcs):
        rdma = pltpu.make_async_remote_copy(
            src_ref=src, dst_ref=src,
            send_sem=send_sems.at[i], recv_sem=recv_sems.at[i],
            device_id=peer, device_id_type=pl.DeviceIdType.MESH,
        )
        rdma.start()
        rdmas.append(rdma)

    q_ref[...] = jnp.dot(x, wq_ref[...], preferred_element_type=F32)
    qr_ref[...] = jnp.dot(x, wqr_ref[...], preferred_element_type=F32)
    kr_ref[...] = jnp.dot(x, wkr_ref[...], preferred_element_type=F32)

    for rdma in rdmas:
        rdma.wait()


def _proj_comm(x2d, wdkv_sh, wuk_sh, wuv_sh, wq, wqr, wkr):
    return pl.pallas_call(
        _proj_comm_body,
        out_shape=(
            jax.ShapeDtypeStruct((S, D), F32),
            jax.ShapeDtypeStruct((S, H * Dr), F32),
            jax.ShapeDtypeStruct((S, Dr), F32),
            jax.ShapeDtypeStruct((S, DC), F32),
            jax.ShapeDtypeStruct((DC, D), F32),
            jax.ShapeDtypeStruct((DC, D), F32),
        ),
        in_specs=[pl.BlockSpec(memory_space=pltpu.VMEM)] * 7,
        out_specs=(pl.BlockSpec(memory_space=pltpu.VMEM),) * 6,
        scratch_shapes=[
            pltpu.SemaphoreType.DMA((3,)),
            pltpu.SemaphoreType.DMA((3,)),
        ],
        compiler_params=pltpu.CompilerParams(
            collective_id=0, vmem_limit_bytes=60 * 1024 * 1024),
    )(x2d, wdkv_sh, wuk_sh, wuv_sh, wq, wqr, wkr)


def _attn_body(q_ref, qr_ref, kr_ref, c_ref, wuk_ref, wuv_ref, wo_ref,
               out_ref):
    h = pl.program_id(0)
    c = c_ref[...]
    k = jnp.dot(c, wuk_ref[...], preferred_element_type=F32)
    v = jnp.dot(c, wuv_ref[...], preferred_element_type=F32)

    s = lax.dot_general(q_ref[...], k, (((1,), (1,)), ((), ())),
                        preferred_element_type=F32)
    s = s + lax.dot_general(qr_ref[0], kr_ref[...],
                            (((1,), (1,)), ((), ())),
                            preferred_element_type=F32)
    p = jnp.exp(s * SCALE)
    denom = jnp.sum(p, axis=1, keepdims=True)
    o = lax.dot_general(p, v, (((1,), (0,)), ((), ())),
                        preferred_element_type=F32)
    o = o / denom
    contrib = jnp.dot(o, wo_ref[...], preferred_element_type=F32)

    @pl.when(h == 0)
    def _():
        out_ref[...] = contrib

    @pl.when(h != 0)
    def _():
        out_ref[...] += contrib


def _attn(q, qr3, kr, c, wukf, wuvf, wo):
    return pl.pallas_call(
        _attn_body,
        grid=(H,),
        in_specs=[
            pl.BlockSpec((S, Dh), lambda h: (0, h)),
            pl.BlockSpec((1, S, Dr), lambda h: (h, 0, 0)),
            pl.BlockSpec((S, Dr), lambda h: (0, 0)),
            pl.BlockSpec((S, DC), lambda h: (0, 0)),
            pl.BlockSpec((DC, Dh), lambda h: (0, h)),
            pl.BlockSpec((DC, Dh), lambda h: (0, h)),
            pl.BlockSpec((Dh, D), lambda h: (h, 0)),
        ],
        out_specs=pl.BlockSpec((S, D), lambda h: (0, 0)),
        out_shape=jax.ShapeDtypeStruct((S, D), F32),
        compiler_params=pltpu.CompilerParams(
            vmem_limit_bytes=60 * 1024 * 1024),
    )(q, qr3, kr, c, wukf, wuvf, wo)


def kernel(x, Wdkv, Wuk, Wuv, Wq, Wqr, Wkr, Wo):
    x2d = x.reshape(S, D)
    q, qr, kr, c, wukf, wuvf = _proj_comm(x2d, Wdkv, Wuk, Wuv, Wq, Wqr, Wkr)
    qr3 = qr.reshape(S, H, Dr).transpose(1, 0, 2)
    out = _attn(q, qr3, kr, c, wukf, wuvf, Wo)
    return out.reshape(B, S, D)


# device time: 97259 ns/iter; 1.6279x vs baseline; 1.5320x over previous
import jax
import jax.numpy as jnp
from jax import lax
from jax.experimental import pallas as pl
from jax.experimental.pallas import tpu as pltpu

B, S, H, Dh, Dr = 1, 1024, 16, 128, 32
D = 2048
DC = 256
DC_SH = 128
SCALE = (Dh + Dr) ** -0.5
F32 = jnp.float32
HPAIR = H // 2


def _proj_comm_body(x_ref, wdkv_ref, wuk_ref, wuv_ref,
                    wq_hbm, wqr_hbm, wkr_hbm,
                    q_ref, qr_ref, kr_ref, c_ref, wukf_ref, wuvf_ref,
                    wq_v, wqr_v, wkr_v, load_sems, send_sems, recv_sems):
    my_x = lax.axis_index("x")
    my_y = lax.axis_index("y")
    peer = (my_x, 1 - my_y)
    off = my_y * DC_SH

    loads = []
    for i, (src, dst) in enumerate(
            [(wq_hbm, wq_v), (wqr_hbm, wqr_v), (wkr_hbm, wkr_v)]):
        cp = pltpu.make_async_copy(src, dst, load_sems.at[i])
        cp.start()
        loads.append(cp)

    x = x_ref[...]
    cpart = jnp.dot(x, wdkv_ref[...], preferred_element_type=F32)
    c_ref[:, pl.ds(off, DC_SH)] = cpart
    wukf_ref[pl.ds(off, DC_SH), :] = wuk_ref[...]
    wuvf_ref[pl.ds(off, DC_SH), :] = wuv_ref[...]

    barrier = pltpu.get_barrier_semaphore()
    pl.semaphore_signal(barrier, inc=1, device_id=peer,
                        device_id_type=pl.DeviceIdType.MESH)
    pl.semaphore_wait(barrier, 1)

    srcs = [
        c_ref.at[:, pl.ds(off, DC_SH)],
        wukf_ref.at[pl.ds(off, DC_SH), :],
        wuvf_ref.at[pl.ds(off, DC_SH), :],
    ]
    rdmas = []
    for i, src in enumerate(srcs):
        rdma = pltpu.make_async_remote_copy(
            src_ref=src, dst_ref=src,
            send_sem=send_sems.at[i], recv_sem=recv_sems.at[i],
            device_id=peer, device_id_type=pl.DeviceIdType.MESH,
        )
        rdma.start()
        rdmas.append(rdma)

    loads[0].wait()
    q_ref[...] = jnp.dot(x, wq_v[...], preferred_element_type=F32)
    loads[1].wait()
    qr_ref[...] = jnp.dot(x, wqr_v[...], preferred_element_type=F32)
    loads[2].wait()
    kr_ref[...] = jnp.dot(x, wkr_v[...], preferred_element_type=F32)

    for rdma in rdmas:
        rdma.wait()


def _proj_comm(x2d, wdkv_sh, wuk_sh, wuv_sh, wq, wqr, wkr):
    return pl.pallas_call(
        _proj_comm_body,
        out_shape=(
            jax.ShapeDtypeStruct((S, D), F32),
            jax.ShapeDtypeStruct((S, H * Dr), F32),
            jax.ShapeDtypeStruct((S, Dr), F32),
            jax.ShapeDtypeStruct((S, DC), F32),
            jax.ShapeDtypeStruct((DC, D), F32),
            jax.ShapeDtypeStruct((DC, D), F32),
        ),
        in_specs=[pl.BlockSpec(memory_space=pltpu.VMEM)] * 4
        + [pl.BlockSpec(memory_space=pl.ANY)] * 3,
        out_specs=(pl.BlockSpec(memory_space=pltpu.VMEM),) * 6,
        scratch_shapes=[
            pltpu.VMEM((D, D), F32),
            pltpu.VMEM((D, H * Dr), F32),
            pltpu.VMEM((D, Dr), F32),
            pltpu.SemaphoreType.DMA((3,)),
            pltpu.SemaphoreType.DMA((3,)),
            pltpu.SemaphoreType.DMA((3,)),
        ],
        compiler_params=pltpu.CompilerParams(
            collective_id=0, vmem_limit_bytes=62 * 1024 * 1024),
    )(x2d, wdkv_sh, wuk_sh, wuv_sh, wq, wqr, wkr)


def _attn_body(q_ref, qr_ref, kr_ref, c_ref, wuk_ref, wuv_ref, wo_ref,
               out_ref, oacc_ref):
    g = pl.program_id(0)
    c = c_ref[...]
    k2 = jnp.dot(c, wuk_ref[...], preferred_element_type=F32)
    v2 = jnp.dot(c, wuv_ref[...], preferred_element_type=F32)
    kr = kr_ref[...]

    for j in range(2):
        qa = jnp.concatenate(
            [q_ref[:, j * Dh:(j + 1) * Dh], qr_ref[j]], axis=1)
        ka = jnp.concatenate(
            [k2[:, j * Dh:(j + 1) * Dh], kr], axis=1)
        s = lax.dot_general(qa, ka, (((1,), (1,)), ((), ())),
                            preferred_element_type=F32)
        p = jnp.exp(s * SCALE)
        denom = jnp.sum(p, axis=1, keepdims=True)
        o = lax.dot_general(p, v2[:, j * Dh:(j + 1) * Dh],
                            (((1,), (0,)), ((), ())),
                            preferred_element_type=F32)
        oacc_ref[:, pl.ds(g * 2 * Dh + j * Dh, Dh)] = o / denom

    @pl.when(g == HPAIR - 1)
    def _():
        out_ref[...] = jnp.dot(oacc_ref[...], wo_ref[...],
                               preferred_element_type=F32)


def _attn(q, qr3, kr, c, wukf, wuvf, wo):
    return pl.pallas_call(
        _attn_body,
        grid=(HPAIR,),
        in_specs=[
            pl.BlockSpec((S, 2 * Dh), lambda g: (0, g)),
            pl.BlockSpec((2, S, Dr), lambda g: (g, 0, 0)),
            pl.BlockSpec((S, Dr), lambda g: (0, 0)),
            pl.BlockSpec((S, DC), lambda g: (0, 0)),
            pl.BlockSpec((DC, 2 * Dh), lambda g: (0, g)),
            pl.BlockSpec((DC, 2 * Dh), lambda g: (0, g)),
            pl.BlockSpec((D, D), lambda g: (0, 0)),
        ],
        out_specs=pl.BlockSpec((S, D), lambda g: (0, 0)),
        out_shape=jax.ShapeDtypeStruct((S, D), F32),
        scratch_shapes=[pltpu.VMEM((S, D), F32)],
        compiler_params=pltpu.CompilerParams(
            vmem_limit_bytes=62 * 1024 * 1024),
    )(q, qr3, kr, c, wukf, wuvf, wo)


def kernel(x, Wdkv, Wuk, Wuv, Wq, Wqr, Wkr, Wo):
    x2d = x.reshape(S, D)
    q, qr, kr, c, wukf, wuvf = _proj_comm(x2d, Wdkv, Wuk, Wuv, Wq, Wqr, Wkr)
    qr3 = qr.reshape(S, H, Dr).transpose(1, 0, 2)
    out = _attn(q, qr3, kr, c, wukf, wuvf, Wo)
    return out.reshape(B, S, D)


# device time: 81217 ns/iter; 1.9494x vs baseline; 1.1975x over previous
import jax
import jax.numpy as jnp
from jax import lax
from jax.experimental import pallas as pl
from jax.experimental.pallas import tpu as pltpu

B, S, H, Dh, Dr = 1, 1024, 16, 128, 32
D = 2048
DC = 256
DC_SH = 128
SCALE = (Dh + Dr) ** -0.5
F32 = jnp.float32
BF16 = jnp.bfloat16
HPAIR = H // 2


def _proj_comm_body(x_ref, wdkv_ref, wuk_ref, wuv_ref,
                    wq_hbm, wqr_hbm, wkr_hbm,
                    q_ref, qr_ref, kr_ref, c_ref, wukf_ref, wuvf_ref,
                    wq_v, wqr_v, wkr_v, load_sems, send_sems, recv_sems):
    my_x = lax.axis_index("x")
    my_y = lax.axis_index("y")
    peer = (my_x, 1 - my_y)
    off = my_y * DC_SH

    ld_qr = pltpu.make_async_copy(wqr_hbm, wqr_v, load_sems.at[0])
    ld_kr = pltpu.make_async_copy(wkr_hbm, wkr_v, load_sems.at[1])
    ld_q = pltpu.make_async_copy(wq_hbm, wq_v, load_sems.at[2])
    ld_qr.start()
    ld_kr.start()
    ld_q.start()

    wukf_ref[pl.ds(off, DC_SH), :] = wuk_ref[...].astype(BF16)
    wuvf_ref[pl.ds(off, DC_SH), :] = wuv_ref[...].astype(BF16)

    barrier = pltpu.get_barrier_semaphore()
    pl.semaphore_signal(barrier, inc=1, device_id=peer,
                        device_id_type=pl.DeviceIdType.MESH)
    pl.semaphore_wait(barrier, 1)

    rdmas = []
    for i, src in enumerate([wukf_ref.at[pl.ds(off, DC_SH), :],
                             wuvf_ref.at[pl.ds(off, DC_SH), :]]):
        rdma = pltpu.make_async_remote_copy(
            src_ref=src, dst_ref=src,
            send_sem=send_sems.at[i], recv_sem=recv_sems.at[i],
            device_id=peer, device_id_type=pl.DeviceIdType.MESH,
        )
        rdma.start()
        rdmas.append(rdma)

    x = x_ref[...]
    cpart = jnp.dot(x, wdkv_ref[...], preferred_element_type=F32)
    c_ref[:, pl.ds(off, DC_SH)] = cpart.astype(BF16)
    c_src = c_ref.at[:, pl.ds(off, DC_SH)]
    rdma_c = pltpu.make_async_remote_copy(
        src_ref=c_src, dst_ref=c_src,
        send_sem=send_sems.at[2], recv_sem=recv_sems.at[2],
        device_id=peer, device_id_type=pl.DeviceIdType.MESH,
    )
    rdma_c.start()
    rdmas.append(rdma_c)

    ld_qr.wait()
    qr_ref[...] = jnp.dot(x, wqr_v[...], preferred_element_type=F32)
    ld_kr.wait()
    kr_ref[...] = jnp.dot(x, wkr_v[...], preferred_element_type=F32)
    ld_q.wait()
    q_ref[...] = jnp.dot(x, wq_v[...], preferred_element_type=F32)

    for rdma in rdmas:
        rdma.wait()


def _proj_comm(x2d, wdkv_sh, wuk_sh, wuv_sh, wq, wqr, wkr):
    return pl.pallas_call(
        _proj_comm_body,
        out_shape=(
            jax.ShapeDtypeStruct((S, D), F32),
            jax.ShapeDtypeStruct((S, H * Dr), F32),
            jax.ShapeDtypeStruct((S, Dr), F32),
            jax.ShapeDtypeStruct((S, DC), BF16),
            jax.ShapeDtypeStruct((DC, D), BF16),
            jax.ShapeDtypeStruct((DC, D), BF16),
        ),
        in_specs=[pl.BlockSpec(memory_space=pltpu.VMEM)] * 4
        + [pl.BlockSpec(memory_space=pl.ANY)] * 3,
        out_specs=(pl.BlockSpec(memory_space=pltpu.VMEM),) * 6,
        scratch_shapes=[
            pltpu.VMEM((D, D), F32),
            pltpu.VMEM((D, H * Dr), F32),
            pltpu.VMEM((D, Dr), F32),
            pltpu.SemaphoreType.DMA((3,)),
            pltpu.SemaphoreType.DMA((3,)),
            pltpu.SemaphoreType.DMA((3,)),
        ],
        compiler_params=pltpu.CompilerParams(
            collective_id=0, vmem_limit_bytes=62 * 1024 * 1024),
    )(x2d, wdkv_sh, wuk_sh, wuv_sh, wq, wqr, wkr)


def _attn_body(q_ref, qr_ref, kr_ref, c_ref, wuk_ref, wuv_ref, wo_hbm,
               out_ref, oacc_ref, wo_v, wo_sem):
    g = pl.program_id(0)

    @pl.when(g == 0)
    def _():
        pltpu.make_async_copy(wo_hbm, wo_v, wo_sem).start()

    c = c_ref[...]
    k2 = jnp.dot(c, wuk_ref[...], preferred_element_type=F32)
    v2 = jnp.dot(c, wuv_ref[...], preferred_element_type=F32)
    kr = kr_ref[...]

    for j in range(2):
        qa = jnp.concatenate(
            [q_ref[:, j * Dh:(j + 1) * Dh], qr_ref[j]], axis=1)
        ka = jnp.concatenate(
            [k2[:, j * Dh:(j + 1) * Dh], kr], axis=1)
        s = lax.dot_general(qa, ka, (((1,), (1,)), ((), ())),
                            preferred_element_type=F32)
        p = jnp.exp(s * SCALE)
        denom = jnp.sum(p, axis=1, keepdims=True)
        o = lax.dot_general(p, v2[:, j * Dh:(j + 1) * Dh],
                            (((1,), (0,)), ((), ())),
                            preferred_element_type=F32)
        oacc_ref[:, pl.ds(g * 2 * Dh + j * Dh, Dh)] = o / denom

    @pl.when(g == HPAIR - 1)
    def _():
        pltpu.make_async_copy(wo_hbm, wo_v, wo_sem).wait()
        out_ref[...] = jnp.dot(oacc_ref[...], wo_v[...],
                               preferred_element_type=F32)


def _attn(q, qr3, kr, c, wukf, wuvf, wo):
    return pl.pallas_call(
        _attn_body,
        grid=(HPAIR,),
        in_specs=[
            pl.BlockSpec((S, 2 * Dh), lambda g: (0, g)),
            pl.BlockSpec((2, S, Dr), lambda g: (g, 0, 0)),
            pl.BlockSpec((S, Dr), lambda g: (0, 0)),
            pl.BlockSpec((S, DC), lambda g: (0, 0)),
            pl.BlockSpec((DC, 2 * Dh), lambda g: (0, g)),
            pl.BlockSpec((DC, 2 * Dh), lambda g: (0, g)),
            pl.BlockSpec(memory_space=pl.ANY),
        ],
        out_specs=pl.BlockSpec((S, D), lambda g: (0, 0)),
        out_shape=jax.ShapeDtypeStruct((S, D), F32),
        scratch_shapes=[
            pltpu.VMEM((S, D), F32),
            pltpu.VMEM((D, D), F32),
            pltpu.SemaphoreType.DMA,
        ],
        compiler_params=pltpu.CompilerParams(
            vmem_limit_bytes=62 * 1024 * 1024),
    )(q, qr3, kr, c, wukf, wuvf, wo)


def kernel(x, Wdkv, Wuk, Wuv, Wq, Wqr, Wkr, Wo):
    x2d = x.reshape(S, D)
    q, qr, kr, c, wukf, wuvf = _proj_comm(x2d, Wdkv, Wuk, Wuv, Wq, Wqr, Wkr)
    qr3 = qr.reshape(S, H, Dr).transpose(1, 0, 2)
    out = _attn(q, qr3, kr, c, wukf, wuvf, Wo)
    return out.reshape(B, S, D)
